# baseline (speedup 1.0000x reference)
import functools

import jax
import jax.numpy as jnp
from jax.experimental import pallas as pl
from jax.experimental.pallas import tpu as pltpu

_VMEM_LIMIT = 100 * 1024 * 1024
_GB = 8


def _fused_kernel(belta_ref, feat_ref, bond_ref, dist_ref, dmask_ref,
                  w1v_ref, w1q_ref, w1k_ref, w1ek_ref, w1ef_hb_ref, b1_ref,
                  sele_ref, selr_ref, selh_ref, w2ekt_ref,
                  w2v_ref, w2q_ref, w2k_ref, w2ef_ref, b2_ref,
                  mw1_ref, mb1_ref, mwa_ref, mw2_ref, mb2_ref,
                  wfc_ref, bfc_ref, o_ref):
    Gb, Nn = feat_ref.shape[0], feat_ref.shape[1]
    rows = Gb * Nn * Nn

    bond8_4d = jnp.swapaxes(bond_ref[...], 1, 2).reshape(Gb, Nn, Nn, 8)
    mask = bond8_4d[:, :, :, 7:8] > 0.0
    bond8 = bond8_4d.reshape(rows, 8)

    feat2 = feat_ref[...].reshape(Gb * Nn, feat_ref.shape[2])
    ft1 = jnp.dot(feat2, w1v_ref[...], preferred_element_type=jnp.float32)
    q1 = jnp.dot(feat2, w1q_ref[...], preferred_element_type=jnp.float32)
    k1 = jnp.dot(feat2, w1k_ref[...], preferred_element_type=jnp.float32)

    w1ek = w1ek_ref[...]
    qk1 = jnp.concatenate([q1, k1], axis=0)
    ab_parts = [
        jnp.dot(qk1 * w1ek[c:c + 1, :], sele_ref[...],
                preferred_element_type=jnp.float32)
        for c in range(8)]
    ab = jnp.concatenate(ab_parts, axis=-1)
    a_i = ab[:Gb * Nn].reshape(Gb, Nn, 1, 64)
    b_j = ab[Gb * Nn:].reshape(Gb, 1, Nn, 64)
    bond_rep = jnp.dot(bond8, selr_ref[...],
                       preferred_element_type=jnp.float32)
    prod = bond_rep.reshape(Gb, Nn, Nn, 64) * (a_i + b_j)
    e1 = jnp.dot(prod.reshape(rows, 64), selh_ref[...],
                 preferred_element_type=jnp.float32).reshape(Gb, Nn, Nn, 8)
    e1 = jnp.where(mask, e1, jnp.float32(-1e30))
    m1 = jnp.max(e1, axis=1, keepdims=True)
    p1 = jnp.where(mask, jnp.exp(e1 - m1), 0.0)
    d1 = jnp.sum(p1, axis=1, keepdims=True)
    a1 = p1 * pl.reciprocal(jnp.maximum(d1, jnp.float32(1e-20)), approx=False)

    ft1_3 = ft1.reshape(Gb, Nn, 256)
    h1_parts = []
    for h in range(8):
        t1h = (a1[:, :, :, h:h + 1] * bond8_4d).reshape(Gb, Nn, Nn * 8)
        m_h = jnp.einsum('gir,gic->grc', t1h,
                         ft1_3[:, :, h * 32:(h + 1) * 32],
                         preferred_element_type=jnp.float32)
        m_h = m_h.reshape(Gb, Nn, 8, 32) * w1ef_hb_ref[h]
        h1_parts.append(jnp.sum(m_h, axis=2))
    h1 = jnp.concatenate(h1_parts, axis=-1)
    h1 = h1 + b1_ref[...]

    h1f = h1.reshape(Gb * Nn, 256)
    ft2 = jnp.dot(h1f, w2v_ref[...], preferred_element_type=jnp.float32)
    q2 = jnp.dot(h1f, w2q_ref[...], preferred_element_type=jnp.float32)
    k2 = jnp.dot(h1f, w2k_ref[...], preferred_element_type=jnp.float32)

    a2i = jnp.dot(q2, w2ekt_ref[...], preferred_element_type=jnp.float32)
    b2j = jnp.dot(k2, w2ekt_ref[...], preferred_element_type=jnp.float32)
    prod2 = bond8_4d * (a2i.reshape(Gb, Nn, 1, 8) + b2j.reshape(Gb, 1, Nn, 8))
    e2 = jnp.dot(prod2.reshape(rows, 8), jnp.ones((8, 1), jnp.float32),
                 preferred_element_type=jnp.float32).reshape(Gb, Nn, Nn, 1)
    e2 = jnp.where(mask, e2, jnp.float32(-1e30))
    m2 = jnp.max(e2, axis=1, keepdims=True)
    p2 = jnp.where(mask, jnp.exp(e2 - m2), 0.0)
    d2 = jnp.sum(p2, axis=1, keepdims=True)
    a2 = p2 * pl.reciprocal(jnp.maximum(d2, jnp.float32(1e-20)), approx=False)
    t2 = (a2 * bond8_4d).reshape(Gb, Nn, Nn * 8)
    m2b = jnp.einsum('gir,gic->grc', t2, ft2.reshape(Gb, Nn, 256),
                     preferred_element_type=jnp.float32)
    h2 = jnp.sum(m2b.reshape(Gb, Nn, 8, 256) * w2ef_ref[...], axis=2)
    h2 = h2 + h1 + b2_ref[...]
    h2 = jnp.where(h2 > 0, h2, jnp.exp(h2) - 1.0)

    h2f = h2.reshape(Gb * Nn, 256)
    fs = jnp.dot(h2f, mw1_ref[...], preferred_element_type=jnp.float32) \
        + mb1_ref[...]
    fs = jnp.where(fs > 0, fs, 0.2 * fs)
    eatt = jnp.dot(h2f, mwa_ref[...], preferred_element_type=jnp.float32)

    dist = dist_ref[...]
    dmask = dmask_ref[...] > 0.0
    belta = belta_ref[0]
    w = jnp.where(dmask,
                  belta * pl.reciprocal(jnp.where(dmask, dist, 1.0),
                                        approx=False), 0.0)
    ftm = jnp.einsum('gds,gsf->gdf', w, fs.reshape(Gb, Nn, 256),
                     preferred_element_type=jnp.float32)
    rst = ftm + eatt.reshape(Gb, Nn, 1) * h2
    mp = jnp.dot(rst.reshape(Gb * Nn, 256), mw2_ref[...],
                 preferred_element_type=jnp.float32) + mb2_ref[...]
    mp = jnp.where(mp > 0, mp, 0.2 * mp)
    mp3 = mp.reshape(Gb, Nn, 256)

    inv_n = jnp.float32(1.0 / Nn)
    sum_g = jnp.sum(h2, axis=1)
    max_g = jnp.max(h2, axis=1)
    sum_m = jnp.sum(mp3, axis=1)
    max_m = jnp.max(mp3, axis=1)
    pooled = jnp.concatenate(
        [sum_g, max_g, sum_m, max_m, sum_m * inv_n, sum_g * inv_n], axis=-1)
    out = jnp.dot(pooled, wfc_ref[...], preferred_element_type=jnp.float32)
    o_ref[0] = out + bfc_ref[...]


def kernel(atom_feature, adj, bond, dist, dist_mask,
           g1_q_w, g1_k_w, g1_v_w, g1_edge_fc_w, g1_edge_fc_b,
           g1_edge_k_w, g1_edge_k_b, g1_bias,
           g2_q_w, g2_k_w, g2_v_w, g2_edge_fc_w, g2_edge_fc_b,
           g2_edge_k_w, g2_edge_k_b, g2_bias,
           mpnn_fc1_w, mpnn_fc1_b, mpnn_fc2_w, mpnn_fc2_b,
           mpnn_attfc_w, mpnn_belta, fc_w, fc_b):
    G, Nn = adj.shape[0], adj.shape[1]
    Fin = atom_feature.shape[1]
    hidden = fc_w.shape[1]
    Gb = _GB if G % _GB == 0 else 1
    NB = G // Gb

    f32 = jnp.float32
    featg = atom_feature.astype(f32).reshape(G, Nn, Fin)

    w1ek = jnp.concatenate([g1_edge_k_w.astype(f32),
                            g1_edge_k_b.astype(f32)[None, :]], axis=0)
    w1ef = jnp.concatenate([g1_edge_fc_w.astype(f32),
                            g1_edge_fc_b.astype(f32)[None, :]], axis=0)
    w2ek = jnp.concatenate([g2_edge_k_w.astype(f32),
                            g2_edge_k_b.astype(f32)[None, :]], axis=0)
    w2ef = jnp.concatenate([g2_edge_fc_w.astype(f32),
                            g2_edge_fc_b.astype(f32)[None, :]], axis=0)

    sel_e = (jax.lax.broadcasted_iota(jnp.int32, (256, 8), 0) // 32 ==
             jax.lax.broadcasted_iota(jnp.int32, (256, 8), 1)).astype(f32)
    sel_o = (jax.lax.broadcasted_iota(jnp.int32, (8, 256), 0) ==
             jax.lax.broadcasted_iota(jnp.int32, (8, 256), 1) // 32).astype(f32)
    sel_r = (jax.lax.broadcasted_iota(jnp.int32, (8, 64), 0) ==
             jax.lax.broadcasted_iota(jnp.int32, (8, 64), 1) // 8).astype(f32)
    sel_h = (jax.lax.broadcasted_iota(jnp.int32, (64, 8), 0) % 8 ==
             jax.lax.broadcasted_iota(jnp.int32, (64, 8), 1)).astype(f32)

    flops = int(G * (2 * Nn * Fin * (3 * 256)
                     + 2 * Nn * Nn * 8 * (2 * 256 + 32 + 256)
                     + 2 * Nn * Nn * (256 * 8 + 8 * 256)
                     + Nn * Nn * (6 * 256 + 3 * 32)
                     + 2 * Nn * 256 * (3 * 256 + 2 * 32)
                     + 4 * Nn * 256 * 256 + 2 * Nn * Nn * 256
                     + 2 * 6 * 256 * hidden + 12 * Nn * 256))
    transc = int(G * Nn * Nn * 9 + 2 * G * Nn * 256)
    bytes_acc = int(4 * (G * Nn * Fin + G * Nn * Nn * 10 + G * hidden
                         + Fin * 768 + 4 * 8 * 256 + 3 * 256 * 256
                         + 6 * 256 * hidden + 256 * 8 * 2 + 1536 * hidden))

    out = pl.pallas_call(
        functools.partial(_fused_kernel),
        out_shape=jax.ShapeDtypeStruct((NB, Gb, hidden), f32),
        grid=(NB,),
        in_specs=[
            pl.BlockSpec(memory_space=pltpu.MemorySpace.SMEM),
            pl.BlockSpec((Gb, Nn, Fin), lambda b: (b, 0, 0)),
            pl.BlockSpec((Gb, 8, Nn * Nn), lambda b: (b, 0, 0)),
            pl.BlockSpec((Gb, Nn, Nn), lambda b: (b, 0, 0)),
            pl.BlockSpec((Gb, Nn, Nn), lambda b: (b, 0, 0)),
            pl.BlockSpec((Fin, 256), lambda b: (0, 0)),
            pl.BlockSpec((Fin, 256), lambda b: (0, 0)),
            pl.BlockSpec((Fin, 256), lambda b: (0, 0)),
            pl.BlockSpec((8, 256), lambda b: (0, 0)),
            pl.BlockSpec((8, 8, 32), lambda b: (0, 0, 0)),
            pl.BlockSpec((1, 256), lambda b: (0, 0)),
            pl.BlockSpec((256, 8), lambda b: (0, 0)),
            pl.BlockSpec((8, 64), lambda b: (0, 0)),
            pl.BlockSpec((64, 8), lambda b: (0, 0)),
            pl.BlockSpec((32, 8), lambda b: (0, 0)),
            pl.BlockSpec((256, 256), lambda b: (0, 0)),
            pl.BlockSpec((256, 32), lambda b: (0, 0)),
            pl.BlockSpec((256, 32), lambda b: (0, 0)),
            pl.BlockSpec((8, 256), lambda b: (0, 0)),
            pl.BlockSpec((1, 256), lambda b: (0, 0)),
            pl.BlockSpec((256, 256), lambda b: (0, 0)),
            pl.BlockSpec((1, 256), lambda b: (0, 0)),
            pl.BlockSpec((256, 1), lambda b: (0, 0)),
            pl.BlockSpec((256, 256), lambda b: (0, 0)),
            pl.BlockSpec((1, 256), lambda b: (0, 0)),
            pl.BlockSpec((6 * 256, hidden), lambda b: (0, 0)),
            pl.BlockSpec((1, hidden), lambda b: (0, 0)),
        ],
        out_specs=pl.BlockSpec((1, Gb, hidden), lambda b: (b, 0, 0)),
        compiler_params=pltpu.CompilerParams(
            dimension_semantics=("parallel",),
            vmem_limit_bytes=_VMEM_LIMIT),
        cost_estimate=pl.CostEstimate(flops=flops, transcendentals=transc,
                                      bytes_accessed=bytes_acc),
    )(mpnn_belta.reshape(1).astype(f32),
      featg,
      jnp.concatenate([bond.astype(f32), adj.astype(f32)[..., None]],
                      axis=-1).transpose(0, 3, 1, 2).reshape(G, 8, Nn * Nn),
      dist.astype(f32),
      dist_mask.astype(f32),
      g1_v_w.astype(f32), g1_q_w.astype(f32), g1_k_w.astype(f32),
      w1ek, w1ef.reshape(8, 8, 32).transpose(1, 0, 2),
      g1_bias.reshape(1, 256).astype(f32),
      sel_e, sel_r, sel_h, w2ek.T,
      g2_v_w.astype(f32), g2_q_w.astype(f32), g2_k_w.astype(f32),
      w2ef, g2_bias.reshape(1, 256).astype(f32),
      mpnn_fc1_w.astype(f32), mpnn_fc1_b.reshape(1, 256).astype(f32),
      mpnn_attfc_w.astype(f32),
      mpnn_fc2_w.astype(f32), mpnn_fc2_b.reshape(1, 256).astype(f32),
      fc_w.astype(f32), fc_b.reshape(1, hidden).astype(f32))

    return out.reshape(G, hidden)


# speedup vs baseline: 1.2506x; 1.2506x over previous
import functools

import jax
import jax.numpy as jnp
from jax.experimental import pallas as pl
from jax.experimental.pallas import tpu as pltpu

_VMEM_LIMIT = 100 * 1024 * 1024
_GB = 8


def _fused_kernel(belta_ref, feat_ref, bond_ref, dist_ref, dmask_ref,
                  w1v_ref, w1q_ref, w1k_ref, w1ek_ref, w1eft_ref, b1_ref,
                  sele_ref, selr_ref, selh_ref, selb_ref, w2ekt_ref,
                  w2v_ref, w2q_ref, w2k_ref, w2eft_ref, b2_ref,
                  mw1_ref, mb1_ref, mwa_ref, mw2_ref, mb2_ref,
                  wfc_ref, bfc_ref, o_ref):
    Gb, Nn = feat_ref.shape[0], feat_ref.shape[1]
    rows = Gb * Nn * Nn

    bond8_4d = jnp.swapaxes(bond_ref[...], 1, 2).reshape(Gb, Nn, Nn, 8)
    mask = bond8_4d[:, :, :, 7:8] > 0.0
    bond8 = bond8_4d.reshape(rows, 8)

    feat2 = feat_ref[...].reshape(Gb * Nn, feat_ref.shape[2])
    ft1 = jnp.dot(feat2, w1v_ref[...], preferred_element_type=jnp.float32)
    q1 = jnp.dot(feat2, w1q_ref[...], preferred_element_type=jnp.float32)
    k1 = jnp.dot(feat2, w1k_ref[...], preferred_element_type=jnp.float32)

    w1ek = w1ek_ref[...]
    qk1 = jnp.concatenate([q1, k1], axis=0)
    ab_parts = [
        jnp.dot(qk1 * w1ek[c:c + 1, :], sele_ref[...],
                preferred_element_type=jnp.float32)
        for c in range(8)]
    ab = jnp.concatenate(ab_parts, axis=-1)
    a_i = ab[:Gb * Nn].reshape(Gb, Nn, 1, 64)
    b_j = ab[Gb * Nn:].reshape(Gb, 1, Nn, 64)
    bond_rep = jnp.dot(bond8, selr_ref[...],
                       preferred_element_type=jnp.float32)
    prod = bond_rep.reshape(Gb, Nn, Nn, 64) * (a_i + b_j)
    e1 = jnp.dot(prod.reshape(rows, 64), selh_ref[...],
                 preferred_element_type=jnp.float32).reshape(Gb, Nn, Nn, 8)
    e1 = jnp.where(mask, e1, jnp.float32(-1e30))
    m1 = jnp.max(e1, axis=1, keepdims=True)
    p1 = jnp.where(mask, jnp.exp(e1 - m1), 0.0)
    d1 = jnp.sum(p1, axis=1, keepdims=True)
    a1 = p1 * pl.reciprocal(jnp.maximum(d1, jnp.float32(1e-20)), approx=False)

    ft1t = jnp.swapaxes(ft1.reshape(Gb, Nn, 256), 1, 2)
    h1t_parts = []
    for h in range(8):
        t1h = (a1[:, :, :, h:h + 1] * bond8_4d).reshape(Gb, Nn, Nn * 8)
        m_h = jnp.einsum('gci,gir->gcr', ft1t[:, h * 32:(h + 1) * 32, :],
                         t1h, preferred_element_type=jnp.float32)
        h1t_parts.append(jnp.dot(
            (m_h * w1eft_ref[h]).reshape(Gb * 32, Nn * 8), selb_ref[...],
            preferred_element_type=jnp.float32).reshape(Gb, 32, Nn))
    h1t = jnp.concatenate(h1t_parts, axis=1)
    h1 = jnp.swapaxes(h1t, 1, 2)
    h1 = h1 + b1_ref[...]

    h1f = h1.reshape(Gb * Nn, 256)
    ft2 = jnp.dot(h1f, w2v_ref[...], preferred_element_type=jnp.float32)
    q2 = jnp.dot(h1f, w2q_ref[...], preferred_element_type=jnp.float32)
    k2 = jnp.dot(h1f, w2k_ref[...], preferred_element_type=jnp.float32)

    a2i = jnp.dot(q2, w2ekt_ref[...], preferred_element_type=jnp.float32)
    b2j = jnp.dot(k2, w2ekt_ref[...], preferred_element_type=jnp.float32)
    prod2 = bond8_4d * (a2i.reshape(Gb, Nn, 1, 8) + b2j.reshape(Gb, 1, Nn, 8))
    e2 = jnp.dot(prod2.reshape(rows, 8), jnp.ones((8, 1), jnp.float32),
                 preferred_element_type=jnp.float32).reshape(Gb, Nn, Nn, 1)
    e2 = jnp.where(mask, e2, jnp.float32(-1e30))
    m2 = jnp.max(e2, axis=1, keepdims=True)
    p2 = jnp.where(mask, jnp.exp(e2 - m2), 0.0)
    d2 = jnp.sum(p2, axis=1, keepdims=True)
    a2 = p2 * pl.reciprocal(jnp.maximum(d2, jnp.float32(1e-20)), approx=False)
    t2 = (a2 * bond8_4d).reshape(Gb, Nn, Nn * 8)
    ft2t = jnp.swapaxes(ft2.reshape(Gb, Nn, 256), 1, 2)
    m2 = jnp.einsum('gci,gir->gcr', ft2t, t2,
                    preferred_element_type=jnp.float32)
    h2t = jnp.dot((m2 * w2eft_ref[...]).reshape(Gb * 256, Nn * 8),
                  selb_ref[...],
                  preferred_element_type=jnp.float32).reshape(Gb, 256, Nn)
    h2 = jnp.swapaxes(h2t, 1, 2)
    h2 = h2 + h1 + b2_ref[...]
    h2 = jnp.where(h2 > 0, h2, jnp.exp(h2) - 1.0)

    h2f = h2.reshape(Gb * Nn, 256)
    fs = jnp.dot(h2f, mw1_ref[...], preferred_element_type=jnp.float32) \
        + mb1_ref[...]
    fs = jnp.where(fs > 0, fs, 0.2 * fs)
    eatt = jnp.dot(h2f, mwa_ref[...], preferred_element_type=jnp.float32)

    dist = dist_ref[...]
    dmask = dmask_ref[...] > 0.0
    belta = belta_ref[0]
    w = jnp.where(dmask,
                  belta * pl.reciprocal(jnp.where(dmask, dist, 1.0),
                                        approx=False), 0.0)
    ftm = jnp.einsum('gds,gsf->gdf', w, fs.reshape(Gb, Nn, 256),
                     preferred_element_type=jnp.float32)
    rst = ftm + eatt.reshape(Gb, Nn, 1) * h2
    mp = jnp.dot(rst.reshape(Gb * Nn, 256), mw2_ref[...],
                 preferred_element_type=jnp.float32) + mb2_ref[...]
    mp = jnp.where(mp > 0, mp, 0.2 * mp)
    mp3 = mp.reshape(Gb, Nn, 256)

    inv_n = jnp.float32(1.0 / Nn)
    sum_g = jnp.sum(h2, axis=1)
    max_g = jnp.max(h2, axis=1)
    sum_m = jnp.sum(mp3, axis=1)
    max_m = jnp.max(mp3, axis=1)
    pooled = jnp.concatenate(
        [sum_g, max_g, sum_m, max_m, sum_m * inv_n, sum_g * inv_n], axis=-1)
    out = jnp.dot(pooled, wfc_ref[...], preferred_element_type=jnp.float32)
    o_ref[0] = out + bfc_ref[...]


def kernel(atom_feature, adj, bond, dist, dist_mask,
           g1_q_w, g1_k_w, g1_v_w, g1_edge_fc_w, g1_edge_fc_b,
           g1_edge_k_w, g1_edge_k_b, g1_bias,
           g2_q_w, g2_k_w, g2_v_w, g2_edge_fc_w, g2_edge_fc_b,
           g2_edge_k_w, g2_edge_k_b, g2_bias,
           mpnn_fc1_w, mpnn_fc1_b, mpnn_fc2_w, mpnn_fc2_b,
           mpnn_attfc_w, mpnn_belta, fc_w, fc_b):
    G, Nn = adj.shape[0], adj.shape[1]
    Fin = atom_feature.shape[1]
    hidden = fc_w.shape[1]
    Gb = _GB if G % _GB == 0 else 1
    NB = G // Gb

    f32 = jnp.float32
    featg = atom_feature.astype(f32).reshape(G, Nn, Fin)

    w1ek = jnp.concatenate([g1_edge_k_w.astype(f32),
                            g1_edge_k_b.astype(f32)[None, :]], axis=0)
    w1ef = jnp.concatenate([g1_edge_fc_w.astype(f32),
                            g1_edge_fc_b.astype(f32)[None, :]], axis=0)
    w2ek = jnp.concatenate([g2_edge_k_w.astype(f32),
                            g2_edge_k_b.astype(f32)[None, :]], axis=0)
    w2ef = jnp.concatenate([g2_edge_fc_w.astype(f32),
                            g2_edge_fc_b.astype(f32)[None, :]], axis=0)

    sel_e = (jax.lax.broadcasted_iota(jnp.int32, (256, 8), 0) // 32 ==
             jax.lax.broadcasted_iota(jnp.int32, (256, 8), 1)).astype(f32)
    sel_b = (jax.lax.broadcasted_iota(jnp.int32, (256, 32), 0) // 8 ==
             jax.lax.broadcasted_iota(jnp.int32, (256, 32), 1)).astype(f32)
    sel_r = (jax.lax.broadcasted_iota(jnp.int32, (8, 64), 0) ==
             jax.lax.broadcasted_iota(jnp.int32, (8, 64), 1) // 8).astype(f32)
    sel_h = (jax.lax.broadcasted_iota(jnp.int32, (64, 8), 0) % 8 ==
             jax.lax.broadcasted_iota(jnp.int32, (64, 8), 1)).astype(f32)

    flops = int(G * (2 * Nn * Fin * (3 * 256)
                     + 2 * Nn * Nn * 8 * (2 * 256 + 32 + 256)
                     + 2 * Nn * Nn * (256 * 8 + 8 * 256)
                     + Nn * Nn * (6 * 256 + 3 * 32)
                     + 2 * Nn * 256 * (3 * 256 + 2 * 32)
                     + 4 * Nn * 256 * 256 + 2 * Nn * Nn * 256
                     + 2 * 6 * 256 * hidden + 12 * Nn * 256))
    transc = int(G * Nn * Nn * 9 + 2 * G * Nn * 256)
    bytes_acc = int(4 * (G * Nn * Fin + G * Nn * Nn * 10 + G * hidden
                         + Fin * 768 + 4 * 8 * 256 + 3 * 256 * 256
                         + 6 * 256 * hidden + 256 * 8 * 2 + 1536 * hidden))

    out = pl.pallas_call(
        functools.partial(_fused_kernel),
        out_shape=jax.ShapeDtypeStruct((NB, Gb, hidden), f32),
        grid=(NB,),
        in_specs=[
            pl.BlockSpec(memory_space=pltpu.MemorySpace.SMEM),
            pl.BlockSpec((Gb, Nn, Fin), lambda b: (b, 0, 0)),
            pl.BlockSpec((Gb, 8, Nn * Nn), lambda b: (b, 0, 0)),
            pl.BlockSpec((Gb, Nn, Nn), lambda b: (b, 0, 0)),
            pl.BlockSpec((Gb, Nn, Nn), lambda b: (b, 0, 0)),
            pl.BlockSpec((Fin, 256), lambda b: (0, 0)),
            pl.BlockSpec((Fin, 256), lambda b: (0, 0)),
            pl.BlockSpec((Fin, 256), lambda b: (0, 0)),
            pl.BlockSpec((8, 256), lambda b: (0, 0)),
            pl.BlockSpec((8, 32, 256), lambda b: (0, 0, 0)),
            pl.BlockSpec((1, 256), lambda b: (0, 0)),
            pl.BlockSpec((256, 8), lambda b: (0, 0)),
            pl.BlockSpec((8, 64), lambda b: (0, 0)),
            pl.BlockSpec((64, 8), lambda b: (0, 0)),
            pl.BlockSpec((256, 32), lambda b: (0, 0)),
            pl.BlockSpec((32, 8), lambda b: (0, 0)),
            pl.BlockSpec((256, 256), lambda b: (0, 0)),
            pl.BlockSpec((256, 32), lambda b: (0, 0)),
            pl.BlockSpec((256, 32), lambda b: (0, 0)),
            pl.BlockSpec((256, 256), lambda b: (0, 0)),
            pl.BlockSpec((1, 256), lambda b: (0, 0)),
            pl.BlockSpec((256, 256), lambda b: (0, 0)),
            pl.BlockSpec((1, 256), lambda b: (0, 0)),
            pl.BlockSpec((256, 1), lambda b: (0, 0)),
            pl.BlockSpec((256, 256), lambda b: (0, 0)),
            pl.BlockSpec((1, 256), lambda b: (0, 0)),
            pl.BlockSpec((6 * 256, hidden), lambda b: (0, 0)),
            pl.BlockSpec((1, hidden), lambda b: (0, 0)),
        ],
        out_specs=pl.BlockSpec((1, Gb, hidden), lambda b: (b, 0, 0)),
        compiler_params=pltpu.CompilerParams(
            dimension_semantics=("parallel",),
            vmem_limit_bytes=_VMEM_LIMIT),
        cost_estimate=pl.CostEstimate(flops=flops, transcendentals=transc,
                                      bytes_accessed=bytes_acc),
    )(mpnn_belta.reshape(1).astype(f32),
      featg,
      jnp.concatenate([bond.astype(f32), adj.astype(f32)[..., None]],
                      axis=-1).transpose(0, 3, 1, 2).reshape(G, 8, Nn * Nn),
      dist.astype(f32),
      dist_mask.astype(f32),
      g1_v_w.astype(f32), g1_q_w.astype(f32), g1_k_w.astype(f32),
      w1ek,
      jnp.broadcast_to(w1ef.reshape(8, 8, 32).transpose(1, 2, 0)[:, :, None, :],
                       (8, 32, 32, 8)).reshape(8, 32, 256),
      g1_bias.reshape(1, 256).astype(f32),
      sel_e, sel_r, sel_h, sel_b, w2ek.T,
      g2_v_w.astype(f32), g2_q_w.astype(f32), g2_k_w.astype(f32),
      jnp.broadcast_to(w2ef.T[:, None, :], (256, 32, 8)).reshape(256, 256),
      g2_bias.reshape(1, 256).astype(f32),
      mpnn_fc1_w.astype(f32), mpnn_fc1_b.reshape(1, 256).astype(f32),
      mpnn_attfc_w.astype(f32),
      mpnn_fc2_w.astype(f32), mpnn_fc2_b.reshape(1, 256).astype(f32),
      fc_w.astype(f32), fc_b.reshape(1, hidden).astype(f32))

    return out.reshape(G, hidden)


# speedup vs baseline: 2.6672x; 2.1327x over previous
import functools

import jax
import jax.numpy as jnp
from jax.experimental import pallas as pl
from jax.experimental.pallas import tpu as pltpu

_VMEM_LIMIT = 100 * 1024 * 1024
_GB = 8


def _fused_kernel(belta_ref, feat_ref, bond_ref, dist_ref, dmask_ref,
                  w1v_ref, w1q_ref, w1k_ref, w1ek_ref, w1ef_ref, b1_ref,
                  sele_ref, selo_ref, selr_ref, selh_ref, w2ekt_ref,
                  w2v_ref, w2q_ref, w2k_ref, w2ef_ref, b2_ref,
                  mw1_ref, mb1_ref, mwa_ref, mw2_ref, mb2_ref,
                  wfc_ref, bfc_ref, o_ref):
    Gb, Nn = feat_ref.shape[0], feat_ref.shape[1]
    rows = Gb * Nn * Nn

    bond8_4d = jnp.swapaxes(bond_ref[...], 1, 2).reshape(Gb, Nn, Nn, 8)
    mask = bond8_4d[:, :, :, 7:8] > 0.0
    bond8 = bond8_4d.reshape(rows, 8)

    feat2 = feat_ref[...].reshape(Gb * Nn, feat_ref.shape[2])
    ft1 = jnp.dot(feat2, w1v_ref[...], preferred_element_type=jnp.float32)
    q1 = jnp.dot(feat2, w1q_ref[...], preferred_element_type=jnp.float32)
    k1 = jnp.dot(feat2, w1k_ref[...], preferred_element_type=jnp.float32)
    bond8_bf = bond8.astype(jnp.bfloat16)
    ef1 = jnp.dot(bond8_bf, w1ef_ref[...],
                  preferred_element_type=jnp.float32).astype(jnp.bfloat16)

    w1ek = w1ek_ref[...]
    qk1 = jnp.concatenate([q1, k1], axis=0)
    ab_parts = [
        jnp.dot(qk1 * w1ek[c:c + 1, :], sele_ref[...],
                preferred_element_type=jnp.float32)
        for c in range(8)]
    ab = jnp.concatenate(ab_parts, axis=-1)
    a_i = ab[:Gb * Nn].reshape(Gb, Nn, 1, 64)
    b_j = ab[Gb * Nn:].reshape(Gb, 1, Nn, 64)
    bond_rep = jnp.dot(bond8, selr_ref[...],
                       preferred_element_type=jnp.float32)
    prod = bond_rep.reshape(Gb, Nn, Nn, 64) * (a_i + b_j)
    e1 = jnp.dot(prod.reshape(rows, 64), selh_ref[...],
                 preferred_element_type=jnp.float32).reshape(Gb, Nn, Nn, 8)
    e1 = jnp.where(mask, e1, jnp.float32(-1e30))
    m1 = jnp.max(e1, axis=1, keepdims=True)
    p1 = jnp.where(mask, jnp.exp(e1 - m1), 0.0)
    d1 = jnp.sum(p1, axis=1, keepdims=True)
    a1 = p1 * pl.reciprocal(jnp.maximum(d1, jnp.float32(1e-20)), approx=False)
    aw1 = jnp.dot(a1.reshape(rows, 8).astype(jnp.bfloat16), selo_ref[...],
                  preferred_element_type=jnp.float32).astype(
                      jnp.bfloat16).reshape(Gb, Nn, Nn, 256)
    h1 = jnp.sum(ft1.reshape(Gb, Nn, 1, 256)
                 * ef1.reshape(Gb, Nn, Nn, 256) * aw1, axis=1)
    h1 = h1 + b1_ref[...]

    h1f = h1.reshape(Gb * Nn, 256)
    ft2 = jnp.dot(h1f, w2v_ref[...], preferred_element_type=jnp.float32)
    q2 = jnp.dot(h1f, w2q_ref[...], preferred_element_type=jnp.float32)
    k2 = jnp.dot(h1f, w2k_ref[...], preferred_element_type=jnp.float32)
    ef2 = jnp.dot(bond8_bf, w2ef_ref[...],
                  preferred_element_type=jnp.float32).astype(jnp.bfloat16)

    a2i = jnp.dot(q2, w2ekt_ref[...], preferred_element_type=jnp.float32)
    b2j = jnp.dot(k2, w2ekt_ref[...], preferred_element_type=jnp.float32)
    prod2 = bond8_4d * (a2i.reshape(Gb, Nn, 1, 8) + b2j.reshape(Gb, 1, Nn, 8))
    e2 = jnp.dot(prod2.reshape(rows, 8), jnp.ones((8, 1), jnp.float32),
                 preferred_element_type=jnp.float32).reshape(Gb, Nn, Nn, 1)
    e2 = jnp.where(mask, e2, jnp.float32(-1e30))
    m2 = jnp.max(e2, axis=1, keepdims=True)
    p2 = jnp.where(mask, jnp.exp(e2 - m2), 0.0)
    d2 = jnp.sum(p2, axis=1, keepdims=True)
    a2 = p2 * pl.reciprocal(jnp.maximum(d2, jnp.float32(1e-20)), approx=False)
    h2 = jnp.sum(ft2.reshape(Gb, Nn, 1, 256)
                 * ef2.reshape(Gb, Nn, Nn, 256) * a2, axis=1)
    h2 = h2 + h1 + b2_ref[...]
    h2 = jnp.where(h2 > 0, h2, jnp.exp(h2) - 1.0)

    h2f = h2.reshape(Gb * Nn, 256)
    fs = jnp.dot(h2f, mw1_ref[...], preferred_element_type=jnp.float32) \
        + mb1_ref[...]
    fs = jnp.where(fs > 0, fs, 0.2 * fs)
    eatt = jnp.dot(h2f, mwa_ref[...], preferred_element_type=jnp.float32)

    dist = dist_ref[...]
    dmask = dmask_ref[...] > 0.0
    belta = belta_ref[0]
    w = jnp.where(dmask,
                  belta * pl.reciprocal(jnp.where(dmask, dist, 1.0),
                                        approx=False), 0.0)
    ftm = jnp.einsum('gds,gsf->gdf', w, fs.reshape(Gb, Nn, 256),
                     preferred_element_type=jnp.float32)
    rst = ftm + eatt.reshape(Gb, Nn, 1) * h2
    mp = jnp.dot(rst.reshape(Gb * Nn, 256), mw2_ref[...],
                 preferred_element_type=jnp.float32) + mb2_ref[...]
    mp = jnp.where(mp > 0, mp, 0.2 * mp)
    mp3 = mp.reshape(Gb, Nn, 256)

    inv_n = jnp.float32(1.0 / Nn)
    sum_g = jnp.sum(h2, axis=1)
    max_g = jnp.max(h2, axis=1)
    sum_m = jnp.sum(mp3, axis=1)
    max_m = jnp.max(mp3, axis=1)
    pooled = jnp.concatenate(
        [sum_g, max_g, sum_m, max_m, sum_m * inv_n, sum_g * inv_n], axis=-1)
    out = jnp.dot(pooled, wfc_ref[...], preferred_element_type=jnp.float32)
    o_ref[0] = out + bfc_ref[...]


def kernel(atom_feature, adj, bond, dist, dist_mask,
           g1_q_w, g1_k_w, g1_v_w, g1_edge_fc_w, g1_edge_fc_b,
           g1_edge_k_w, g1_edge_k_b, g1_bias,
           g2_q_w, g2_k_w, g2_v_w, g2_edge_fc_w, g2_edge_fc_b,
           g2_edge_k_w, g2_edge_k_b, g2_bias,
           mpnn_fc1_w, mpnn_fc1_b, mpnn_fc2_w, mpnn_fc2_b,
           mpnn_attfc_w, mpnn_belta, fc_w, fc_b):
    G, Nn = adj.shape[0], adj.shape[1]
    Fin = atom_feature.shape[1]
    hidden = fc_w.shape[1]
    Gb = _GB if G % _GB == 0 else 1
    NB = G // Gb

    f32 = jnp.float32
    featg = atom_feature.astype(f32).reshape(G, Nn, Fin)

    w1ek = jnp.concatenate([g1_edge_k_w.astype(f32),
                            g1_edge_k_b.astype(f32)[None, :]], axis=0)
    w1ef = jnp.concatenate([g1_edge_fc_w.astype(f32),
                            g1_edge_fc_b.astype(f32)[None, :]], axis=0)
    w2ek = jnp.concatenate([g2_edge_k_w.astype(f32),
                            g2_edge_k_b.astype(f32)[None, :]], axis=0)
    w2ef = jnp.concatenate([g2_edge_fc_w.astype(f32),
                            g2_edge_fc_b.astype(f32)[None, :]], axis=0)

    sel_e = (jax.lax.broadcasted_iota(jnp.int32, (256, 8), 0) // 32 ==
             jax.lax.broadcasted_iota(jnp.int32, (256, 8), 1)).astype(f32)
    sel_o = (jax.lax.broadcasted_iota(jnp.int32, (8, 256), 0) ==
             jax.lax.broadcasted_iota(jnp.int32, (8, 256), 1) // 32).astype(f32)
    sel_r = (jax.lax.broadcasted_iota(jnp.int32, (8, 64), 0) ==
             jax.lax.broadcasted_iota(jnp.int32, (8, 64), 1) // 8).astype(f32)
    sel_h = (jax.lax.broadcasted_iota(jnp.int32, (64, 8), 0) % 8 ==
             jax.lax.broadcasted_iota(jnp.int32, (64, 8), 1)).astype(f32)

    flops = int(G * (2 * Nn * Fin * (3 * 256)
                     + 2 * Nn * Nn * 8 * (2 * 256 + 32 + 256)
                     + 2 * Nn * Nn * (256 * 8 + 8 * 256)
                     + Nn * Nn * (6 * 256 + 3 * 32)
                     + 2 * Nn * 256 * (3 * 256 + 2 * 32)
                     + 4 * Nn * 256 * 256 + 2 * Nn * Nn * 256
                     + 2 * 6 * 256 * hidden + 12 * Nn * 256))
    transc = int(G * Nn * Nn * 9 + 2 * G * Nn * 256)
    bytes_acc = int(4 * (G * Nn * Fin + G * Nn * Nn * 10 + G * hidden
                         + Fin * 768 + 4 * 8 * 256 + 3 * 256 * 256
                         + 6 * 256 * hidden + 256 * 8 * 2 + 1536 * hidden))

    out = pl.pallas_call(
        functools.partial(_fused_kernel),
        out_shape=jax.ShapeDtypeStruct((NB, Gb, hidden), f32),
        grid=(NB,),
        in_specs=[
            pl.BlockSpec(memory_space=pltpu.MemorySpace.SMEM),
            pl.BlockSpec((Gb, Nn, Fin), lambda b: (b, 0, 0)),
            pl.BlockSpec((Gb, 8, Nn * Nn), lambda b: (b, 0, 0)),
            pl.BlockSpec((Gb, Nn, Nn), lambda b: (b, 0, 0)),
            pl.BlockSpec((Gb, Nn, Nn), lambda b: (b, 0, 0)),
            pl.BlockSpec((Fin, 256), lambda b: (0, 0)),
            pl.BlockSpec((Fin, 256), lambda b: (0, 0)),
            pl.BlockSpec((Fin, 256), lambda b: (0, 0)),
            pl.BlockSpec((8, 256), lambda b: (0, 0)),
            pl.BlockSpec((8, 256), lambda b: (0, 0)),
            pl.BlockSpec((1, 256), lambda b: (0, 0)),
            pl.BlockSpec((256, 8), lambda b: (0, 0)),
            pl.BlockSpec((8, 256), lambda b: (0, 0)),
            pl.BlockSpec((8, 64), lambda b: (0, 0)),
            pl.BlockSpec((64, 8), lambda b: (0, 0)),
            pl.BlockSpec((32, 8), lambda b: (0, 0)),
            pl.BlockSpec((256, 256), lambda b: (0, 0)),
            pl.BlockSpec((256, 32), lambda b: (0, 0)),
            pl.BlockSpec((256, 32), lambda b: (0, 0)),
            pl.BlockSpec((8, 256), lambda b: (0, 0)),
            pl.BlockSpec((1, 256), lambda b: (0, 0)),
            pl.BlockSpec((256, 256), lambda b: (0, 0)),
            pl.BlockSpec((1, 256), lambda b: (0, 0)),
            pl.BlockSpec((256, 1), lambda b: (0, 0)),
            pl.BlockSpec((256, 256), lambda b: (0, 0)),
            pl.BlockSpec((1, 256), lambda b: (0, 0)),
            pl.BlockSpec((6 * 256, hidden), lambda b: (0, 0)),
            pl.BlockSpec((1, hidden), lambda b: (0, 0)),
        ],
        out_specs=pl.BlockSpec((1, Gb, hidden), lambda b: (b, 0, 0)),
        compiler_params=pltpu.CompilerParams(
            dimension_semantics=("parallel",),
            vmem_limit_bytes=_VMEM_LIMIT),
        cost_estimate=pl.CostEstimate(flops=flops, transcendentals=transc,
                                      bytes_accessed=bytes_acc),
    )(mpnn_belta.reshape(1).astype(f32),
      featg,
      jnp.concatenate([bond.astype(f32), adj.astype(f32)[..., None]],
                      axis=-1).transpose(0, 3, 1, 2).reshape(G, 8, Nn * Nn),
      dist.astype(f32),
      dist_mask.astype(f32),
      g1_v_w.astype(f32), g1_q_w.astype(f32), g1_k_w.astype(f32),
      w1ek, w1ef.astype(jnp.bfloat16), g1_bias.reshape(1, 256).astype(f32),
      sel_e, sel_o.astype(jnp.bfloat16), sel_r, sel_h, w2ek.T,
      g2_v_w.astype(f32), g2_q_w.astype(f32), g2_k_w.astype(f32),
      w2ef.astype(jnp.bfloat16), g2_bias.reshape(1, 256).astype(f32),
      mpnn_fc1_w.astype(f32), mpnn_fc1_b.reshape(1, 256).astype(f32),
      mpnn_attfc_w.astype(f32),
      mpnn_fc2_w.astype(f32), mpnn_fc2_b.reshape(1, 256).astype(f32),
      fc_w.astype(f32), fc_b.reshape(1, hidden).astype(f32))

    return out.reshape(G, hidden)


# speedup vs baseline: 2.9137x; 1.0924x over previous
import functools

import jax
import jax.numpy as jnp
from jax.experimental import pallas as pl
from jax.experimental.pallas import tpu as pltpu

_VMEM_LIMIT = 100 * 1024 * 1024
_GB = 16


def _fused_kernel(belta_ref, feat_ref, bond_ref, dist_ref, dmask_ref,
                  w1v_ref, w1q_ref, w1k_ref, w1ek_ref, w1ef_ref, b1_ref,
                  sele_ref, selo_ref, selr_ref, selh_ref, w2ekt_ref,
                  w2v_ref, w2q_ref, w2k_ref, w2ef_ref, b2_ref,
                  mw1_ref, mb1_ref, mwa_ref, mw2_ref, mb2_ref,
                  wfc_ref, bfc_ref, o_ref):
    Gb, Nn = feat_ref.shape[0], feat_ref.shape[1]
    rows = Gb * Nn * Nn

    bond8_4d = jnp.swapaxes(bond_ref[...], 1, 2).reshape(Gb, Nn, Nn, 8)
    mask = bond8_4d[:, :, :, 7:8] > 0.0
    bond8 = bond8_4d.reshape(rows, 8)

    feat2 = feat_ref[...].reshape(Gb * Nn, feat_ref.shape[2])
    ft1 = jnp.dot(feat2, w1v_ref[...], preferred_element_type=jnp.float32)
    q1 = jnp.dot(feat2, w1q_ref[...], preferred_element_type=jnp.float32)
    k1 = jnp.dot(feat2, w1k_ref[...], preferred_element_type=jnp.float32)
    ef1 = jnp.dot(bond8, w1ef_ref[...], preferred_element_type=jnp.float32)

    w1ek = w1ek_ref[...]
    qk1 = jnp.concatenate([q1, k1], axis=0)
    ab_parts = [
        jnp.dot(qk1 * w1ek[c:c + 1, :], sele_ref[...],
                preferred_element_type=jnp.float32)
        for c in range(8)]
    ab = jnp.concatenate(ab_parts, axis=-1)
    a_i = ab[:Gb * Nn].reshape(Gb, Nn, 1, 64)
    b_j = ab[Gb * Nn:].reshape(Gb, 1, Nn, 64)
    bond_rep = jnp.dot(bond8, selr_ref[...],
                       preferred_element_type=jnp.float32)
    prod = bond_rep.reshape(Gb, Nn, Nn, 64) * (a_i + b_j)
    e1 = jnp.dot(prod.reshape(rows, 64), selh_ref[...],
                 preferred_element_type=jnp.float32).reshape(Gb, Nn, Nn, 8)
    e1 = jnp.where(mask, e1, jnp.float32(-1e30))
    m1 = jnp.max(e1, axis=1, keepdims=True)
    p1 = jnp.where(mask, jnp.exp(e1 - m1), 0.0)
    d1 = jnp.sum(p1, axis=1, keepdims=True)
    a1 = p1 * pl.reciprocal(jnp.maximum(d1, jnp.float32(1e-20)), approx=False)
    aw1 = jnp.dot(a1.reshape(rows, 8), selo_ref[...],
                  preferred_element_type=jnp.float32).reshape(Gb, Nn, Nn, 256)
    h1 = jnp.sum(ft1.reshape(Gb, Nn, 1, 256)
                 * ef1.reshape(Gb, Nn, Nn, 256) * aw1, axis=1)
    h1 = h1 + b1_ref[...]

    h1f = h1.reshape(Gb * Nn, 256)
    ft2 = jnp.dot(h1f, w2v_ref[...], preferred_element_type=jnp.float32)
    q2 = jnp.dot(h1f, w2q_ref[...], preferred_element_type=jnp.float32)
    k2 = jnp.dot(h1f, w2k_ref[...], preferred_element_type=jnp.float32)
    ef2 = jnp.dot(bond8, w2ef_ref[...], preferred_element_type=jnp.float32)

    a2i = jnp.dot(q2, w2ekt_ref[...], preferred_element_type=jnp.float32)
    b2j = jnp.dot(k2, w2ekt_ref[...], preferred_element_type=jnp.float32)
    prod2 = bond8_4d * (a2i.reshape(Gb, Nn, 1, 8) + b2j.reshape(Gb, 1, Nn, 8))
    e2 = jnp.dot(prod2.reshape(rows, 8), jnp.ones((8, 1), jnp.float32),
                 preferred_element_type=jnp.float32).reshape(Gb, Nn, Nn, 1)
    e2 = jnp.where(mask, e2, jnp.float32(-1e30))
    m2 = jnp.max(e2, axis=1, keepdims=True)
    p2 = jnp.where(mask, jnp.exp(e2 - m2), 0.0)
    d2 = jnp.sum(p2, axis=1, keepdims=True)
    a2 = p2 * pl.reciprocal(jnp.maximum(d2, jnp.float32(1e-20)), approx=False)
    h2 = jnp.sum(ft2.reshape(Gb, Nn, 1, 256)
                 * ef2.reshape(Gb, Nn, Nn, 256) * a2, axis=1)
    h2 = h2 + h1 + b2_ref[...]
    h2 = jnp.where(h2 > 0, h2, jnp.exp(h2) - 1.0)

    h2f = h2.reshape(Gb * Nn, 256)
    fs = jnp.dot(h2f, mw1_ref[...], preferred_element_type=jnp.float32) \
        + mb1_ref[...]
    fs = jnp.where(fs > 0, fs, 0.2 * fs)
    eatt = jnp.dot(h2f, mwa_ref[...], preferred_element_type=jnp.float32)

    dist = dist_ref[...]
    dmask = dmask_ref[...] > 0.0
    belta = belta_ref[0]
    w = jnp.where(dmask,
                  belta * pl.reciprocal(jnp.where(dmask, dist, 1.0),
                                        approx=False), 0.0)
    ftm = jnp.einsum('gds,gsf->gdf', w, fs.reshape(Gb, Nn, 256),
                     preferred_element_type=jnp.float32)
    rst = ftm + eatt.reshape(Gb, Nn, 1) * h2
    mp = jnp.dot(rst.reshape(Gb * Nn, 256), mw2_ref[...],
                 preferred_element_type=jnp.float32) + mb2_ref[...]
    mp = jnp.where(mp > 0, mp, 0.2 * mp)
    mp3 = mp.reshape(Gb, Nn, 256)

    inv_n = jnp.float32(1.0 / Nn)
    sum_g = jnp.sum(h2, axis=1)
    max_g = jnp.max(h2, axis=1)
    sum_m = jnp.sum(mp3, axis=1)
    max_m = jnp.max(mp3, axis=1)
    pooled = jnp.concatenate(
        [sum_g, max_g, sum_m, max_m, sum_m * inv_n, sum_g * inv_n], axis=-1)
    out = jnp.dot(pooled, wfc_ref[...], preferred_element_type=jnp.float32)
    o_ref[0] = out + bfc_ref[...]


def kernel(atom_feature, adj, bond, dist, dist_mask,
           g1_q_w, g1_k_w, g1_v_w, g1_edge_fc_w, g1_edge_fc_b,
           g1_edge_k_w, g1_edge_k_b, g1_bias,
           g2_q_w, g2_k_w, g2_v_w, g2_edge_fc_w, g2_edge_fc_b,
           g2_edge_k_w, g2_edge_k_b, g2_bias,
           mpnn_fc1_w, mpnn_fc1_b, mpnn_fc2_w, mpnn_fc2_b,
           mpnn_attfc_w, mpnn_belta, fc_w, fc_b):
    G, Nn = adj.shape[0], adj.shape[1]
    Fin = atom_feature.shape[1]
    hidden = fc_w.shape[1]
    Gb = _GB if G % _GB == 0 else 1
    NB = G // Gb

    f32 = jnp.float32
    featg = atom_feature.astype(f32).reshape(G, Nn, Fin)

    w1ek = jnp.concatenate([g1_edge_k_w.astype(f32),
                            g1_edge_k_b.astype(f32)[None, :]], axis=0)
    w1ef = jnp.concatenate([g1_edge_fc_w.astype(f32),
                            g1_edge_fc_b.astype(f32)[None, :]], axis=0)
    w2ek = jnp.concatenate([g2_edge_k_w.astype(f32),
                            g2_edge_k_b.astype(f32)[None, :]], axis=0)
    w2ef = jnp.concatenate([g2_edge_fc_w.astype(f32),
                            g2_edge_fc_b.astype(f32)[None, :]], axis=0)

    sel_e = (jax.lax.broadcasted_iota(jnp.int32, (256, 8), 0) // 32 ==
             jax.lax.broadcasted_iota(jnp.int32, (256, 8), 1)).astype(f32)
    sel_o = (jax.lax.broadcasted_iota(jnp.int32, (8, 256), 0) ==
             jax.lax.broadcasted_iota(jnp.int32, (8, 256), 1) // 32).astype(f32)
    sel_r = (jax.lax.broadcasted_iota(jnp.int32, (8, 64), 0) ==
             jax.lax.broadcasted_iota(jnp.int32, (8, 64), 1) // 8).astype(f32)
    sel_h = (jax.lax.broadcasted_iota(jnp.int32, (64, 8), 0) % 8 ==
             jax.lax.broadcasted_iota(jnp.int32, (64, 8), 1)).astype(f32)

    flops = int(G * (2 * Nn * Fin * (3 * 256)
                     + 2 * Nn * Nn * 8 * (2 * 256 + 32 + 256)
                     + 2 * Nn * Nn * (256 * 8 + 8 * 256)
                     + Nn * Nn * (6 * 256 + 3 * 32)
                     + 2 * Nn * 256 * (3 * 256 + 2 * 32)
                     + 4 * Nn * 256 * 256 + 2 * Nn * Nn * 256
                     + 2 * 6 * 256 * hidden + 12 * Nn * 256))
    transc = int(G * Nn * Nn * 9 + 2 * G * Nn * 256)
    bytes_acc = int(4 * (G * Nn * Fin + G * Nn * Nn * 10 + G * hidden
                         + Fin * 768 + 4 * 8 * 256 + 3 * 256 * 256
                         + 6 * 256 * hidden + 256 * 8 * 2 + 1536 * hidden))

    out = pl.pallas_call(
        functools.partial(_fused_kernel),
        out_shape=jax.ShapeDtypeStruct((NB, Gb, hidden), f32),
        grid=(NB,),
        in_specs=[
            pl.BlockSpec(memory_space=pltpu.MemorySpace.SMEM),
            pl.BlockSpec((Gb, Nn, Fin), lambda b: (b, 0, 0)),
            pl.BlockSpec((Gb, 8, Nn * Nn), lambda b: (b, 0, 0)),
            pl.BlockSpec((Gb, Nn, Nn), lambda b: (b, 0, 0)),
            pl.BlockSpec((Gb, Nn, Nn), lambda b: (b, 0, 0)),
            pl.BlockSpec((Fin, 256), lambda b: (0, 0)),
            pl.BlockSpec((Fin, 256), lambda b: (0, 0)),
            pl.BlockSpec((Fin, 256), lambda b: (0, 0)),
            pl.BlockSpec((8, 256), lambda b: (0, 0)),
            pl.BlockSpec((8, 256), lambda b: (0, 0)),
            pl.BlockSpec((1, 256), lambda b: (0, 0)),
            pl.BlockSpec((256, 8), lambda b: (0, 0)),
            pl.BlockSpec((8, 256), lambda b: (0, 0)),
            pl.BlockSpec((8, 64), lambda b: (0, 0)),
            pl.BlockSpec((64, 8), lambda b: (0, 0)),
            pl.BlockSpec((32, 8), lambda b: (0, 0)),
            pl.BlockSpec((256, 256), lambda b: (0, 0)),
            pl.BlockSpec((256, 32), lambda b: (0, 0)),
            pl.BlockSpec((256, 32), lambda b: (0, 0)),
            pl.BlockSpec((8, 256), lambda b: (0, 0)),
            pl.BlockSpec((1, 256), lambda b: (0, 0)),
            pl.BlockSpec((256, 256), lambda b: (0, 0)),
            pl.BlockSpec((1, 256), lambda b: (0, 0)),
            pl.BlockSpec((256, 1), lambda b: (0, 0)),
            pl.BlockSpec((256, 256), lambda b: (0, 0)),
            pl.BlockSpec((1, 256), lambda b: (0, 0)),
            pl.BlockSpec((6 * 256, hidden), lambda b: (0, 0)),
            pl.BlockSpec((1, hidden), lambda b: (0, 0)),
        ],
        out_specs=pl.BlockSpec((1, Gb, hidden), lambda b: (b, 0, 0)),
        compiler_params=pltpu.CompilerParams(
            dimension_semantics=("parallel",),
            vmem_limit_bytes=_VMEM_LIMIT),
        cost_estimate=pl.CostEstimate(flops=flops, transcendentals=transc,
                                      bytes_accessed=bytes_acc),
    )(mpnn_belta.reshape(1).astype(f32),
      featg,
      jnp.concatenate([bond.astype(f32), adj.astype(f32)[..., None]],
                      axis=-1).transpose(0, 3, 1, 2).reshape(G, 8, Nn * Nn),
      dist.astype(f32),
      dist_mask.astype(f32),
      g1_v_w.astype(f32), g1_q_w.astype(f32), g1_k_w.astype(f32),
      w1ek, w1ef, g1_bias.reshape(1, 256).astype(f32),
      sel_e, sel_o, sel_r, sel_h, w2ek.T,
      g2_v_w.astype(f32), g2_q_w.astype(f32), g2_k_w.astype(f32),
      w2ef, g2_bias.reshape(1, 256).astype(f32),
      mpnn_fc1_w.astype(f32), mpnn_fc1_b.reshape(1, 256).astype(f32),
      mpnn_attfc_w.astype(f32),
      mpnn_fc2_w.astype(f32), mpnn_fc2_b.reshape(1, 256).astype(f32),
      fc_w.astype(f32), fc_b.reshape(1, hidden).astype(f32))

    return out.reshape(G, hidden)


# speedup vs baseline: 3.0694x; 1.0534x over previous
import functools

import jax
import jax.numpy as jnp
from jax.experimental import pallas as pl
from jax.experimental.pallas import tpu as pltpu

_VMEM_LIMIT = 100 * 1024 * 1024
_GB = 16


def _fused_kernel(belta_ref, feat_ref, bond_ref, dist_ref, dmask_ref,
                  w1v_ref, w1q_ref, w1k_ref, w1ek_ref, w1ef_ref, b1_ref,
                  sele_ref, selo_ref, selr_ref, selh_ref, w2ekt_ref,
                  w2v_ref, w2q_ref, w2k_ref, w2ef_ref, b2_ref,
                  mw1_ref, mb1_ref, mwa_ref, mw2_ref, mb2_ref,
                  wfc_ref, bfc_ref, o_ref):
    Gb, Nn = feat_ref.shape[0], feat_ref.shape[1]
    rows = Gb * Nn * Nn

    bond8_4d = jnp.swapaxes(bond_ref[...], 1, 2).reshape(Gb, Nn, Nn, 8)
    mask = bond8_4d[:, :, :, 7:8] > 0.0
    bond8 = bond8_4d.reshape(rows, 8)

    feat2 = feat_ref[...].reshape(Gb * Nn, feat_ref.shape[2])
    ft1 = jnp.dot(feat2, w1v_ref[...], preferred_element_type=jnp.float32)
    q1 = jnp.dot(feat2, w1q_ref[...], preferred_element_type=jnp.float32)
    k1 = jnp.dot(feat2, w1k_ref[...], preferred_element_type=jnp.float32)
    ef1 = jnp.dot(bond8, w1ef_ref[...], preferred_element_type=jnp.float32)

    w1ek = w1ek_ref[...]
    qk1 = jnp.concatenate([q1, k1], axis=0)
    ab_parts = [
        jnp.dot(qk1 * w1ek[c:c + 1, :], sele_ref[...],
                preferred_element_type=jnp.float32)
        for c in range(8)]
    ab = jnp.concatenate(ab_parts, axis=-1)
    a_i = ab[:Gb * Nn].reshape(Gb, Nn, 1, 64)
    b_j = ab[Gb * Nn:].reshape(Gb, 1, Nn, 64)
    bond_rep = jnp.dot(bond8, selr_ref[...],
                       preferred_element_type=jnp.float32)
    prod = bond_rep.reshape(Gb, Nn, Nn, 64) * (a_i + b_j)
    e1 = jnp.dot(prod.reshape(rows, 64), selh_ref[...],
                 preferred_element_type=jnp.float32).reshape(Gb, Nn, Nn, 8)
    e1 = jnp.where(mask, e1, jnp.float32(-1e30))
    m1 = jnp.max(e1, axis=1, keepdims=True)
    p1 = jnp.exp(e1 - m1)
    d1 = jnp.sum(p1, axis=1, keepdims=True)
    a1 = p1 * pl.reciprocal(jnp.maximum(d1, jnp.float32(1e-20)), approx=False)
    aw1 = jnp.dot(a1.reshape(rows, 8), selo_ref[...],
                  preferred_element_type=jnp.float32).reshape(Gb, Nn, Nn, 256)
    h1 = jnp.sum(ft1.reshape(Gb, Nn, 1, 256)
                 * ef1.reshape(Gb, Nn, Nn, 256) * aw1, axis=1)
    h1 = h1 + b1_ref[...]

    h1f = h1.reshape(Gb * Nn, 256)
    ft2 = jnp.dot(h1f, w2v_ref[...], preferred_element_type=jnp.float32)
    q2 = jnp.dot(h1f, w2q_ref[...], preferred_element_type=jnp.float32)
    k2 = jnp.dot(h1f, w2k_ref[...], preferred_element_type=jnp.float32)
    ef2 = jnp.dot(bond8, w2ef_ref[...], preferred_element_type=jnp.float32)

    a2i = jnp.dot(q2, w2ekt_ref[...], preferred_element_type=jnp.float32)
    b2j = jnp.dot(k2, w2ekt_ref[...], preferred_element_type=jnp.float32)
    prod2 = bond8_4d * (a2i.reshape(Gb, Nn, 1, 8) + b2j.reshape(Gb, 1, Nn, 8))
    e2 = jnp.dot(prod2.reshape(rows, 8), jnp.ones((8, 1), jnp.float32),
                 preferred_element_type=jnp.float32).reshape(Gb, Nn, Nn, 1)
    e2 = jnp.where(mask, e2, jnp.float32(-1e30))
    m2 = jnp.max(e2, axis=1, keepdims=True)
    p2 = jnp.exp(e2 - m2)
    d2 = jnp.sum(p2, axis=1, keepdims=True)
    a2 = p2 * pl.reciprocal(jnp.maximum(d2, jnp.float32(1e-20)), approx=False)
    h2 = jnp.sum(ft2.reshape(Gb, Nn, 1, 256)
                 * ef2.reshape(Gb, Nn, Nn, 256) * a2, axis=1)
    h2 = h2 + h1 + b2_ref[...]
    h2 = jnp.where(h2 > 0, h2, jnp.exp(h2) - 1.0)

    h2f = h2.reshape(Gb * Nn, 256)
    fs = jnp.dot(h2f, mw1_ref[...], preferred_element_type=jnp.float32) \
        + mb1_ref[...]
    fs = jnp.where(fs > 0, fs, 0.2 * fs)
    eatt = jnp.dot(h2f, mwa_ref[...], preferred_element_type=jnp.float32)

    dist = dist_ref[...]
    dmask = dmask_ref[...] > 0.0
    belta = belta_ref[0]
    w = jnp.where(dmask,
                  belta * pl.reciprocal(jnp.where(dmask, dist, 1.0),
                                        approx=False), 0.0)
    ftm = jnp.einsum('gds,gsf->gdf', w, fs.reshape(Gb, Nn, 256),
                     preferred_element_type=jnp.float32)
    rst = ftm + eatt.reshape(Gb, Nn, 1) * h2
    mp = jnp.dot(rst.reshape(Gb * Nn, 256), mw2_ref[...],
                 preferred_element_type=jnp.float32) + mb2_ref[...]
    mp = jnp.where(mp > 0, mp, 0.2 * mp)
    mp3 = mp.reshape(Gb, Nn, 256)

    inv_n = jnp.float32(1.0 / Nn)
    sum_g = jnp.sum(h2, axis=1)
    max_g = jnp.max(h2, axis=1)
    sum_m = jnp.sum(mp3, axis=1)
    max_m = jnp.max(mp3, axis=1)
    pooled = jnp.concatenate(
        [sum_g, max_g, sum_m, max_m, sum_m * inv_n, sum_g * inv_n], axis=-1)
    out = jnp.dot(pooled, wfc_ref[...], preferred_element_type=jnp.float32)
    o_ref[0] = out + bfc_ref[...]


def kernel(atom_feature, adj, bond, dist, dist_mask,
           g1_q_w, g1_k_w, g1_v_w, g1_edge_fc_w, g1_edge_fc_b,
           g1_edge_k_w, g1_edge_k_b, g1_bias,
           g2_q_w, g2_k_w, g2_v_w, g2_edge_fc_w, g2_edge_fc_b,
           g2_edge_k_w, g2_edge_k_b, g2_bias,
           mpnn_fc1_w, mpnn_fc1_b, mpnn_fc2_w, mpnn_fc2_b,
           mpnn_attfc_w, mpnn_belta, fc_w, fc_b):
    G, Nn = adj.shape[0], adj.shape[1]
    Fin = atom_feature.shape[1]
    hidden = fc_w.shape[1]
    Gb = _GB if G % _GB == 0 else 1
    NB = G // Gb

    f32 = jnp.float32
    featg = atom_feature.astype(f32).reshape(G, Nn, Fin)

    w1ek = jnp.concatenate([g1_edge_k_w.astype(f32),
                            g1_edge_k_b.astype(f32)[None, :]], axis=0)
    w1ef = jnp.concatenate([g1_edge_fc_w.astype(f32),
                            g1_edge_fc_b.astype(f32)[None, :]], axis=0)
    w2ek = jnp.concatenate([g2_edge_k_w.astype(f32),
                            g2_edge_k_b.astype(f32)[None, :]], axis=0)
    w2ef = jnp.concatenate([g2_edge_fc_w.astype(f32),
                            g2_edge_fc_b.astype(f32)[None, :]], axis=0)

    sel_e = (jax.lax.broadcasted_iota(jnp.int32, (256, 8), 0) // 32 ==
             jax.lax.broadcasted_iota(jnp.int32, (256, 8), 1)).astype(f32)
    sel_o = (jax.lax.broadcasted_iota(jnp.int32, (8, 256), 0) ==
             jax.lax.broadcasted_iota(jnp.int32, (8, 256), 1) // 32).astype(f32)
    sel_r = (jax.lax.broadcasted_iota(jnp.int32, (8, 64), 0) ==
             jax.lax.broadcasted_iota(jnp.int32, (8, 64), 1) // 8).astype(f32)
    sel_h = (jax.lax.broadcasted_iota(jnp.int32, (64, 8), 0) % 8 ==
             jax.lax.broadcasted_iota(jnp.int32, (64, 8), 1)).astype(f32)

    flops = int(G * (2 * Nn * Fin * (3 * 256)
                     + 2 * Nn * Nn * 8 * (2 * 256 + 32 + 256)
                     + 2 * Nn * Nn * (256 * 8 + 8 * 256)
                     + Nn * Nn * (6 * 256 + 3 * 32)
                     + 2 * Nn * 256 * (3 * 256 + 2 * 32)
                     + 4 * Nn * 256 * 256 + 2 * Nn * Nn * 256
                     + 2 * 6 * 256 * hidden + 12 * Nn * 256))
    transc = int(G * Nn * Nn * 9 + 2 * G * Nn * 256)
    bytes_acc = int(4 * (G * Nn * Fin + G * Nn * Nn * 10 + G * hidden
                         + Fin * 768 + 4 * 8 * 256 + 3 * 256 * 256
                         + 6 * 256 * hidden + 256 * 8 * 2 + 1536 * hidden))

    out = pl.pallas_call(
        functools.partial(_fused_kernel),
        out_shape=jax.ShapeDtypeStruct((NB, Gb, hidden), f32),
        grid=(NB,),
        in_specs=[
            pl.BlockSpec(memory_space=pltpu.MemorySpace.SMEM),
            pl.BlockSpec((Gb, Nn, Fin), lambda b: (b, 0, 0)),
            pl.BlockSpec((Gb, 8, Nn * Nn), lambda b: (b, 0, 0)),
            pl.BlockSpec((Gb, Nn, Nn), lambda b: (b, 0, 0)),
            pl.BlockSpec((Gb, Nn, Nn), lambda b: (b, 0, 0)),
            pl.BlockSpec((Fin, 256), lambda b: (0, 0)),
            pl.BlockSpec((Fin, 256), lambda b: (0, 0)),
            pl.BlockSpec((Fin, 256), lambda b: (0, 0)),
            pl.BlockSpec((8, 256), lambda b: (0, 0)),
            pl.BlockSpec((8, 256), lambda b: (0, 0)),
            pl.BlockSpec((1, 256), lambda b: (0, 0)),
            pl.BlockSpec((256, 8), lambda b: (0, 0)),
            pl.BlockSpec((8, 256), lambda b: (0, 0)),
            pl.BlockSpec((8, 64), lambda b: (0, 0)),
            pl.BlockSpec((64, 8), lambda b: (0, 0)),
            pl.BlockSpec((32, 8), lambda b: (0, 0)),
            pl.BlockSpec((256, 256), lambda b: (0, 0)),
            pl.BlockSpec((256, 32), lambda b: (0, 0)),
            pl.BlockSpec((256, 32), lambda b: (0, 0)),
            pl.BlockSpec((8, 256), lambda b: (0, 0)),
            pl.BlockSpec((1, 256), lambda b: (0, 0)),
            pl.BlockSpec((256, 256), lambda b: (0, 0)),
            pl.BlockSpec((1, 256), lambda b: (0, 0)),
            pl.BlockSpec((256, 1), lambda b: (0, 0)),
            pl.BlockSpec((256, 256), lambda b: (0, 0)),
            pl.BlockSpec((1, 256), lambda b: (0, 0)),
            pl.BlockSpec((6 * 256, hidden), lambda b: (0, 0)),
            pl.BlockSpec((1, hidden), lambda b: (0, 0)),
        ],
        out_specs=pl.BlockSpec((1, Gb, hidden), lambda b: (b, 0, 0)),
        compiler_params=pltpu.CompilerParams(
            dimension_semantics=("parallel",),
            vmem_limit_bytes=_VMEM_LIMIT),
        cost_estimate=pl.CostEstimate(flops=flops, transcendentals=transc,
                                      bytes_accessed=bytes_acc),
    )(mpnn_belta.reshape(1).astype(f32),
      featg,
      jnp.concatenate([bond.astype(f32), adj.astype(f32)[..., None]],
                      axis=-1).transpose(0, 3, 1, 2).reshape(G, 8, Nn * Nn),
      dist.astype(f32),
      dist_mask.astype(f32),
      g1_v_w.astype(f32), g1_q_w.astype(f32), g1_k_w.astype(f32),
      w1ek, w1ef, g1_bias.reshape(1, 256).astype(f32),
      sel_e, sel_o, sel_r, sel_h, w2ek.T,
      g2_v_w.astype(f32), g2_q_w.astype(f32), g2_k_w.astype(f32),
      w2ef, g2_bias.reshape(1, 256).astype(f32),
      mpnn_fc1_w.astype(f32), mpnn_fc1_b.reshape(1, 256).astype(f32),
      mpnn_attfc_w.astype(f32),
      mpnn_fc2_w.astype(f32), mpnn_fc2_b.reshape(1, 256).astype(f32),
      fc_w.astype(f32), fc_b.reshape(1, hidden).astype(f32))

    return out.reshape(G, hidden)


# speedup vs baseline: 3.1931x; 1.0403x over previous
import functools

import jax
import jax.numpy as jnp
from jax.experimental import pallas as pl
from jax.experimental.pallas import tpu as pltpu

_VMEM_LIMIT = 100 * 1024 * 1024
_GB = 16


def _fused_kernel(belta_ref, feat_ref, bond_ref, dist_ref, dmask_ref,
                  w1v_ref, w1q_ref, w1k_ref, w1ef_ref, b1_ref,
                  wblk_ref, selo_ref, selr_ref, selh_ref, w2ekt_ref,
                  w2v_ref, w2q_ref, w2k_ref, w2ef_ref, b2_ref,
                  mw1_ref, mb1_ref, mwa_ref, mw2_ref, mb2_ref,
                  wfc_ref, bfc_ref, o_ref):
    Gb, Nn = feat_ref.shape[0], feat_ref.shape[1]
    rows = Gb * Nn * Nn

    bond8_4d = jnp.swapaxes(bond_ref[...], 1, 2).reshape(Gb, Nn, Nn, 8)
    mask = bond8_4d[:, :, :, 7:8] > 0.0
    bond8 = bond8_4d.reshape(rows, 8)

    feat2 = feat_ref[...].reshape(Gb * Nn, feat_ref.shape[2])
    ft1 = jnp.dot(feat2, w1v_ref[...], preferred_element_type=jnp.float32)
    q1 = jnp.dot(feat2, w1q_ref[...], preferred_element_type=jnp.float32)
    k1 = jnp.dot(feat2, w1k_ref[...], preferred_element_type=jnp.float32)
    ef1 = jnp.dot(bond8, w1ef_ref[...], preferred_element_type=jnp.float32)

    qk1 = jnp.concatenate([q1, k1], axis=0)
    ab = jnp.dot(qk1, wblk_ref[...],
                 preferred_element_type=jnp.float32)
    a_i = ab[:Gb * Nn].reshape(Gb, Nn, 1, 64)
    b_j = ab[Gb * Nn:].reshape(Gb, 1, Nn, 64)
    bond_rep = jnp.dot(bond8, selr_ref[...],
                       preferred_element_type=jnp.float32)
    prod = bond_rep.reshape(Gb, Nn, Nn, 64) * (a_i + b_j)
    e1 = jnp.dot(prod.reshape(rows, 64), selh_ref[...],
                 preferred_element_type=jnp.float32).reshape(Gb, Nn, Nn, 8)
    e1 = jnp.where(mask, e1, jnp.float32(-1e30))
    m1 = jnp.max(e1, axis=1, keepdims=True)
    p1 = jnp.exp(e1 - m1)
    d1 = jnp.sum(p1, axis=1, keepdims=True)
    a1 = p1 * pl.reciprocal(jnp.maximum(d1, jnp.float32(1e-20)), approx=False)
    aw1 = jnp.dot(a1.reshape(rows, 8), selo_ref[...],
                  preferred_element_type=jnp.float32).reshape(Gb, Nn, Nn, 256)
    h1 = jnp.sum(ft1.reshape(Gb, Nn, 1, 256)
                 * ef1.reshape(Gb, Nn, Nn, 256) * aw1, axis=1)
    h1 = h1 + b1_ref[...]

    h1f = h1.reshape(Gb * Nn, 256)
    ft2 = jnp.dot(h1f, w2v_ref[...], preferred_element_type=jnp.float32)
    q2 = jnp.dot(h1f, w2q_ref[...], preferred_element_type=jnp.float32)
    k2 = jnp.dot(h1f, w2k_ref[...], preferred_element_type=jnp.float32)
    ef2 = jnp.dot(bond8, w2ef_ref[...], preferred_element_type=jnp.float32)

    a2i = jnp.dot(q2, w2ekt_ref[...], preferred_element_type=jnp.float32)
    b2j = jnp.dot(k2, w2ekt_ref[...], preferred_element_type=jnp.float32)
    prod2 = bond8_4d * (a2i.reshape(Gb, Nn, 1, 8) + b2j.reshape(Gb, 1, Nn, 8))
    e2 = jnp.dot(prod2.reshape(rows, 8), jnp.ones((8, 1), jnp.float32),
                 preferred_element_type=jnp.float32).reshape(Gb, Nn, Nn, 1)
    e2 = jnp.where(mask, e2, jnp.float32(-1e30))
    m2 = jnp.max(e2, axis=1, keepdims=True)
    p2 = jnp.exp(e2 - m2)
    d2 = jnp.sum(p2, axis=1, keepdims=True)
    a2 = p2 * pl.reciprocal(jnp.maximum(d2, jnp.float32(1e-20)), approx=False)
    h2 = jnp.sum(ft2.reshape(Gb, Nn, 1, 256)
                 * ef2.reshape(Gb, Nn, Nn, 256) * a2, axis=1)
    h2 = h2 + h1 + b2_ref[...]
    h2 = jnp.where(h2 > 0, h2, jnp.exp(h2) - 1.0)

    h2f = h2.reshape(Gb * Nn, 256)
    fs = jnp.dot(h2f, mw1_ref[...], preferred_element_type=jnp.float32) \
        + mb1_ref[...]
    fs = jnp.where(fs > 0, fs, 0.2 * fs)
    eatt = jnp.dot(h2f, mwa_ref[...], preferred_element_type=jnp.float32)

    dist = dist_ref[...]
    dmask = dmask_ref[...] > 0.0
    belta = belta_ref[0]
    w = jnp.where(dmask,
                  belta * pl.reciprocal(jnp.where(dmask, dist, 1.0),
                                        approx=False), 0.0)
    ftm = jnp.einsum('gds,gsf->gdf', w, fs.reshape(Gb, Nn, 256),
                     preferred_element_type=jnp.float32)
    rst = ftm + eatt.reshape(Gb, Nn, 1) * h2
    mp = jnp.dot(rst.reshape(Gb * Nn, 256), mw2_ref[...],
                 preferred_element_type=jnp.float32) + mb2_ref[...]
    mp = jnp.where(mp > 0, mp, 0.2 * mp)
    mp3 = mp.reshape(Gb, Nn, 256)

    inv_n = jnp.float32(1.0 / Nn)
    sum_g = jnp.sum(h2, axis=1)
    max_g = jnp.max(h2, axis=1)
    sum_m = jnp.sum(mp3, axis=1)
    max_m = jnp.max(mp3, axis=1)
    pooled = jnp.concatenate(
        [sum_g, max_g, sum_m, max_m, sum_m * inv_n, sum_g * inv_n], axis=-1)
    out = jnp.dot(pooled, wfc_ref[...], preferred_element_type=jnp.float32)
    o_ref[0] = out + bfc_ref[...]


def kernel(atom_feature, adj, bond, dist, dist_mask,
           g1_q_w, g1_k_w, g1_v_w, g1_edge_fc_w, g1_edge_fc_b,
           g1_edge_k_w, g1_edge_k_b, g1_bias,
           g2_q_w, g2_k_w, g2_v_w, g2_edge_fc_w, g2_edge_fc_b,
           g2_edge_k_w, g2_edge_k_b, g2_bias,
           mpnn_fc1_w, mpnn_fc1_b, mpnn_fc2_w, mpnn_fc2_b,
           mpnn_attfc_w, mpnn_belta, fc_w, fc_b):
    G, Nn = adj.shape[0], adj.shape[1]
    Fin = atom_feature.shape[1]
    hidden = fc_w.shape[1]
    Gb = _GB if G % _GB == 0 else 1
    NB = G // Gb

    f32 = jnp.float32
    featg = atom_feature.astype(f32).reshape(G, Nn, Fin)

    w1ek = jnp.concatenate([g1_edge_k_w.astype(f32),
                            g1_edge_k_b.astype(f32)[None, :]], axis=0)
    w1ef = jnp.concatenate([g1_edge_fc_w.astype(f32),
                            g1_edge_fc_b.astype(f32)[None, :]], axis=0)
    w2ek = jnp.concatenate([g2_edge_k_w.astype(f32),
                            g2_edge_k_b.astype(f32)[None, :]], axis=0)
    w2ef = jnp.concatenate([g2_edge_fc_w.astype(f32),
                            g2_edge_fc_b.astype(f32)[None, :]], axis=0)

    headmask = (jax.lax.broadcasted_iota(jnp.int32, (256, 64), 0) // 32 ==
                jax.lax.broadcasted_iota(jnp.int32, (256, 64), 1) % 8).astype(f32)
    wblk = jnp.repeat(w1ek.T, 8, axis=1) * headmask
    sel_o = (jax.lax.broadcasted_iota(jnp.int32, (8, 256), 0) ==
             jax.lax.broadcasted_iota(jnp.int32, (8, 256), 1) // 32).astype(f32)
    sel_r = (jax.lax.broadcasted_iota(jnp.int32, (8, 64), 0) ==
             jax.lax.broadcasted_iota(jnp.int32, (8, 64), 1) // 8).astype(f32)
    sel_h = (jax.lax.broadcasted_iota(jnp.int32, (64, 8), 0) % 8 ==
             jax.lax.broadcasted_iota(jnp.int32, (64, 8), 1)).astype(f32)

    flops = int(G * (2 * Nn * Fin * (3 * 256)
                     + 2 * Nn * Nn * 8 * (2 * 256 + 32 + 256)
                     + 2 * Nn * Nn * (256 * 8 + 8 * 256)
                     + Nn * Nn * (6 * 256 + 3 * 32)
                     + 2 * Nn * 256 * (3 * 256 + 2 * 32)
                     + 4 * Nn * 256 * 256 + 2 * Nn * Nn * 256
                     + 2 * 6 * 256 * hidden + 12 * Nn * 256))
    transc = int(G * Nn * Nn * 9 + 2 * G * Nn * 256)
    bytes_acc = int(4 * (G * Nn * Fin + G * Nn * Nn * 10 + G * hidden
                         + Fin * 768 + 4 * 8 * 256 + 3 * 256 * 256
                         + 6 * 256 * hidden + 256 * 8 * 2 + 1536 * hidden))

    out = pl.pallas_call(
        functools.partial(_fused_kernel),
        out_shape=jax.ShapeDtypeStruct((NB, Gb, hidden), f32),
        grid=(NB,),
        in_specs=[
            pl.BlockSpec(memory_space=pltpu.MemorySpace.SMEM),
            pl.BlockSpec((Gb, Nn, Fin), lambda b: (b, 0, 0)),
            pl.BlockSpec((Gb, 8, Nn * Nn), lambda b: (b, 0, 0)),
            pl.BlockSpec((Gb, Nn, Nn), lambda b: (b, 0, 0)),
            pl.BlockSpec((Gb, Nn, Nn), lambda b: (b, 0, 0)),
            pl.BlockSpec((Fin, 256), lambda b: (0, 0)),
            pl.BlockSpec((Fin, 256), lambda b: (0, 0)),
            pl.BlockSpec((Fin, 256), lambda b: (0, 0)),
            pl.BlockSpec((8, 256), lambda b: (0, 0)),
            pl.BlockSpec((1, 256), lambda b: (0, 0)),
            pl.BlockSpec((256, 64), lambda b: (0, 0)),
            pl.BlockSpec((8, 256), lambda b: (0, 0)),
            pl.BlockSpec((8, 64), lambda b: (0, 0)),
            pl.BlockSpec((64, 8), lambda b: (0, 0)),
            pl.BlockSpec((32, 8), lambda b: (0, 0)),
            pl.BlockSpec((256, 256), lambda b: (0, 0)),
            pl.BlockSpec((256, 32), lambda b: (0, 0)),
            pl.BlockSpec((256, 32), lambda b: (0, 0)),
            pl.BlockSpec((8, 256), lambda b: (0, 0)),
            pl.BlockSpec((1, 256), lambda b: (0, 0)),
            pl.BlockSpec((256, 256), lambda b: (0, 0)),
            pl.BlockSpec((1, 256), lambda b: (0, 0)),
            pl.BlockSpec((256, 1), lambda b: (0, 0)),
            pl.BlockSpec((256, 256), lambda b: (0, 0)),
            pl.BlockSpec((1, 256), lambda b: (0, 0)),
            pl.BlockSpec((6 * 256, hidden), lambda b: (0, 0)),
            pl.BlockSpec((1, hidden), lambda b: (0, 0)),
        ],
        out_specs=pl.BlockSpec((1, Gb, hidden), lambda b: (b, 0, 0)),
        compiler_params=pltpu.CompilerParams(
            dimension_semantics=("parallel",),
            vmem_limit_bytes=_VMEM_LIMIT),
        cost_estimate=pl.CostEstimate(flops=flops, transcendentals=transc,
                                      bytes_accessed=bytes_acc),
    )(mpnn_belta.reshape(1).astype(f32),
      featg,
      jnp.concatenate([bond.astype(f32), adj.astype(f32)[..., None]],
                      axis=-1).transpose(0, 3, 1, 2).reshape(G, 8, Nn * Nn),
      dist.astype(f32),
      dist_mask.astype(f32),
      g1_v_w.astype(f32), g1_q_w.astype(f32), g1_k_w.astype(f32),
      w1ef, g1_bias.reshape(1, 256).astype(f32),
      wblk, sel_o, sel_r, sel_h, w2ek.T,
      g2_v_w.astype(f32), g2_q_w.astype(f32), g2_k_w.astype(f32),
      w2ef, g2_bias.reshape(1, 256).astype(f32),
      mpnn_fc1_w.astype(f32), mpnn_fc1_b.reshape(1, 256).astype(f32),
      mpnn_attfc_w.astype(f32),
      mpnn_fc2_w.astype(f32), mpnn_fc2_b.reshape(1, 256).astype(f32),
      fc_w.astype(f32), fc_b.reshape(1, hidden).astype(f32))

    return out.reshape(G, hidden)


# speedup vs baseline: 3.2986x; 1.0330x over previous
import functools

import jax
import jax.numpy as jnp
from jax.experimental import pallas as pl
from jax.experimental.pallas import tpu as pltpu

_VMEM_LIMIT = 100 * 1024 * 1024
_GB = 16


def _fused_kernel(belta_ref, feat_ref, bond_ref, dist_ref, dmask_ref,
                  w1v_ref, wqkab_ref, w1ef_ref, b1_ref,
                  selo_ref, selr_ref, selh_ref,
                  w2v_ref, wqk2_ref, w2ef_ref, b2_ref,
                  mw1_ref, mb1_ref, mwa_ref, mw2_ref, mb2_ref,
                  wfc_ref, bfc_ref, o_ref):
    Gb, Nn = feat_ref.shape[0], feat_ref.shape[1]
    rows = Gb * Nn * Nn

    bond8_4d = jnp.swapaxes(bond_ref[...], 1, 2).reshape(Gb, Nn, Nn, 8)
    mask = bond8_4d[:, :, :, 7:8] > 0.0
    bond8 = bond8_4d.reshape(rows, 8)

    feat2 = feat_ref[...].reshape(Gb * Nn, feat_ref.shape[2])
    ft1 = jnp.dot(feat2, w1v_ref[...], preferred_element_type=jnp.float32)
    ef1 = jnp.dot(bond8, w1ef_ref[...], preferred_element_type=jnp.float32)

    ab = jnp.dot(feat2, wqkab_ref[...],
                 preferred_element_type=jnp.float32)
    a_i = ab[:, :64].reshape(Gb, Nn, 1, 64)
    b_j = ab[:, 64:].reshape(Gb, 1, Nn, 64)
    bond_rep = jnp.dot(bond8, selr_ref[...],
                       preferred_element_type=jnp.float32)
    prod = bond_rep.reshape(Gb, Nn, Nn, 64) * (a_i + b_j)
    e1 = jnp.dot(prod.reshape(rows, 64), selh_ref[...],
                 preferred_element_type=jnp.float32).reshape(Gb, Nn, Nn, 8)
    e1 = jnp.where(mask, e1, jnp.float32(-1e30))
    m1 = jnp.max(e1, axis=1, keepdims=True)
    p1 = jnp.exp(e1 - m1)
    d1 = jnp.sum(p1, axis=1, keepdims=True)
    a1 = p1 * pl.reciprocal(jnp.maximum(d1, jnp.float32(1e-20)), approx=False)
    aw1 = jnp.dot(a1.reshape(rows, 8), selo_ref[...],
                  preferred_element_type=jnp.float32).reshape(Gb, Nn, Nn, 256)
    h1 = jnp.sum(ft1.reshape(Gb, Nn, 1, 256)
                 * ef1.reshape(Gb, Nn, Nn, 256) * aw1, axis=1)
    h1 = h1 + b1_ref[...]

    h1f = h1.reshape(Gb * Nn, 256)
    ft2 = jnp.dot(h1f, w2v_ref[...], preferred_element_type=jnp.float32)
    ef2 = jnp.dot(bond8, w2ef_ref[...], preferred_element_type=jnp.float32)

    qk2 = jnp.dot(h1f, wqk2_ref[...], preferred_element_type=jnp.float32)
    prod2 = bond8_4d * (qk2[:, :8].reshape(Gb, Nn, 1, 8)
                        + qk2[:, 8:].reshape(Gb, 1, Nn, 8))
    e2 = jnp.dot(prod2.reshape(rows, 8), jnp.ones((8, 1), jnp.float32),
                 preferred_element_type=jnp.float32).reshape(Gb, Nn, Nn, 1)
    e2 = jnp.where(mask, e2, jnp.float32(-1e30))
    m2 = jnp.max(e2, axis=1, keepdims=True)
    p2 = jnp.exp(e2 - m2)
    d2 = jnp.sum(p2, axis=1, keepdims=True)
    a2 = p2 * pl.reciprocal(jnp.maximum(d2, jnp.float32(1e-20)), approx=False)
    h2 = jnp.sum(ft2.reshape(Gb, Nn, 1, 256)
                 * ef2.reshape(Gb, Nn, Nn, 256) * a2, axis=1)
    h2 = h2 + h1 + b2_ref[...]
    h2 = jnp.where(h2 > 0, h2, jnp.exp(h2) - 1.0)

    h2f = h2.reshape(Gb * Nn, 256)
    fs = jnp.dot(h2f, mw1_ref[...], preferred_element_type=jnp.float32) \
        + mb1_ref[...]
    fs = jnp.where(fs > 0, fs, 0.2 * fs)
    eatt = jnp.dot(h2f, mwa_ref[...], preferred_element_type=jnp.float32)

    dist = dist_ref[...]
    dmask = dmask_ref[...] > 0.0
    belta = belta_ref[0]
    w = jnp.where(dmask,
                  belta * pl.reciprocal(jnp.where(dmask, dist, 1.0),
                                        approx=False), 0.0)
    ftm = jnp.einsum('gds,gsf->gdf', w, fs.reshape(Gb, Nn, 256),
                     preferred_element_type=jnp.float32)
    rst = ftm + eatt.reshape(Gb, Nn, 1) * h2
    mp = jnp.dot(rst.reshape(Gb * Nn, 256), mw2_ref[...],
                 preferred_element_type=jnp.float32) + mb2_ref[...]
    mp = jnp.where(mp > 0, mp, 0.2 * mp)
    mp3 = mp.reshape(Gb, Nn, 256)

    inv_n = jnp.float32(1.0 / Nn)
    sum_g = jnp.sum(h2, axis=1)
    max_g = jnp.max(h2, axis=1)
    sum_m = jnp.sum(mp3, axis=1)
    max_m = jnp.max(mp3, axis=1)
    pooled = jnp.concatenate(
        [sum_g, max_g, sum_m, max_m, sum_m * inv_n, sum_g * inv_n], axis=-1)
    out = jnp.dot(pooled, wfc_ref[...], preferred_element_type=jnp.float32)
    o_ref[0] = out + bfc_ref[...]


def kernel(atom_feature, adj, bond, dist, dist_mask,
           g1_q_w, g1_k_w, g1_v_w, g1_edge_fc_w, g1_edge_fc_b,
           g1_edge_k_w, g1_edge_k_b, g1_bias,
           g2_q_w, g2_k_w, g2_v_w, g2_edge_fc_w, g2_edge_fc_b,
           g2_edge_k_w, g2_edge_k_b, g2_bias,
           mpnn_fc1_w, mpnn_fc1_b, mpnn_fc2_w, mpnn_fc2_b,
           mpnn_attfc_w, mpnn_belta, fc_w, fc_b):
    G, Nn = adj.shape[0], adj.shape[1]
    Fin = atom_feature.shape[1]
    hidden = fc_w.shape[1]
    Gb = _GB if G % _GB == 0 else 1
    NB = G // Gb

    f32 = jnp.float32
    featg = atom_feature.astype(f32).reshape(G, Nn, Fin)

    w1ek = jnp.concatenate([g1_edge_k_w.astype(f32),
                            g1_edge_k_b.astype(f32)[None, :]], axis=0)
    w1ef = jnp.concatenate([g1_edge_fc_w.astype(f32),
                            g1_edge_fc_b.astype(f32)[None, :]], axis=0)
    w2ek = jnp.concatenate([g2_edge_k_w.astype(f32),
                            g2_edge_k_b.astype(f32)[None, :]], axis=0)
    w2ef = jnp.concatenate([g2_edge_fc_w.astype(f32),
                            g2_edge_fc_b.astype(f32)[None, :]], axis=0)

    headmask = (jax.lax.broadcasted_iota(jnp.int32, (256, 64), 0) // 32 ==
                jax.lax.broadcasted_iota(jnp.int32, (256, 64), 1) % 8).astype(f32)
    wblk = jnp.repeat(w1ek.T, 8, axis=1) * headmask
    sel_o = (jax.lax.broadcasted_iota(jnp.int32, (8, 256), 0) ==
             jax.lax.broadcasted_iota(jnp.int32, (8, 256), 1) // 32).astype(f32)
    sel_r = (jax.lax.broadcasted_iota(jnp.int32, (8, 64), 0) ==
             jax.lax.broadcasted_iota(jnp.int32, (8, 64), 1) // 8).astype(f32)
    sel_h = (jax.lax.broadcasted_iota(jnp.int32, (64, 8), 0) % 8 ==
             jax.lax.broadcasted_iota(jnp.int32, (64, 8), 1)).astype(f32)

    flops = int(G * (2 * Nn * Fin * (3 * 256)
                     + 2 * Nn * Nn * 8 * (2 * 256 + 32 + 256)
                     + 2 * Nn * Nn * (256 * 8 + 8 * 256)
                     + Nn * Nn * (6 * 256 + 3 * 32)
                     + 2 * Nn * 256 * (3 * 256 + 2 * 32)
                     + 4 * Nn * 256 * 256 + 2 * Nn * Nn * 256
                     + 2 * 6 * 256 * hidden + 12 * Nn * 256))
    transc = int(G * Nn * Nn * 9 + 2 * G * Nn * 256)
    bytes_acc = int(4 * (G * Nn * Fin + G * Nn * Nn * 10 + G * hidden
                         + Fin * 768 + 4 * 8 * 256 + 3 * 256 * 256
                         + 6 * 256 * hidden + 256 * 8 * 2 + 1536 * hidden))

    out = pl.pallas_call(
        functools.partial(_fused_kernel),
        out_shape=jax.ShapeDtypeStruct((NB, Gb, hidden), f32),
        grid=(NB,),
        in_specs=[
            pl.BlockSpec(memory_space=pltpu.MemorySpace.SMEM),
            pl.BlockSpec((Gb, Nn, Fin), lambda b: (b, 0, 0)),
            pl.BlockSpec((Gb, 8, Nn * Nn), lambda b: (b, 0, 0)),
            pl.BlockSpec((Gb, Nn, Nn), lambda b: (b, 0, 0)),
            pl.BlockSpec((Gb, Nn, Nn), lambda b: (b, 0, 0)),
            pl.BlockSpec((Fin, 256), lambda b: (0, 0)),
            pl.BlockSpec((Fin, 128), lambda b: (0, 0)),
            pl.BlockSpec((8, 256), lambda b: (0, 0)),
            pl.BlockSpec((1, 256), lambda b: (0, 0)),
            pl.BlockSpec((8, 256), lambda b: (0, 0)),
            pl.BlockSpec((8, 64), lambda b: (0, 0)),
            pl.BlockSpec((64, 8), lambda b: (0, 0)),
            pl.BlockSpec((256, 256), lambda b: (0, 0)),
            pl.BlockSpec((256, 16), lambda b: (0, 0)),
            pl.BlockSpec((8, 256), lambda b: (0, 0)),
            pl.BlockSpec((1, 256), lambda b: (0, 0)),
            pl.BlockSpec((256, 256), lambda b: (0, 0)),
            pl.BlockSpec((1, 256), lambda b: (0, 0)),
            pl.BlockSpec((256, 1), lambda b: (0, 0)),
            pl.BlockSpec((256, 256), lambda b: (0, 0)),
            pl.BlockSpec((1, 256), lambda b: (0, 0)),
            pl.BlockSpec((6 * 256, hidden), lambda b: (0, 0)),
            pl.BlockSpec((1, hidden), lambda b: (0, 0)),
        ],
        out_specs=pl.BlockSpec((1, Gb, hidden), lambda b: (b, 0, 0)),
        compiler_params=pltpu.CompilerParams(
            dimension_semantics=("parallel",),
            vmem_limit_bytes=_VMEM_LIMIT),
        cost_estimate=pl.CostEstimate(flops=flops, transcendentals=transc,
                                      bytes_accessed=bytes_acc),
    )(mpnn_belta.reshape(1).astype(f32),
      featg,
      jnp.concatenate([bond.astype(f32), adj.astype(f32)[..., None]],
                      axis=-1).transpose(0, 3, 1, 2).reshape(G, 8, Nn * Nn),
      dist.astype(f32),
      dist_mask.astype(f32),
      g1_v_w.astype(f32),
      jnp.concatenate([g1_q_w.astype(f32) @ wblk,
                       g1_k_w.astype(f32) @ wblk], axis=1),
      w1ef, g1_bias.reshape(1, 256).astype(f32),
      sel_o, sel_r, sel_h,
      g2_v_w.astype(f32),
      jnp.concatenate([g2_q_w.astype(f32) @ w2ek.T,
                       g2_k_w.astype(f32) @ w2ek.T], axis=1),
      w2ef, g2_bias.reshape(1, 256).astype(f32),
      mpnn_fc1_w.astype(f32), mpnn_fc1_b.reshape(1, 256).astype(f32),
      mpnn_attfc_w.astype(f32),
      mpnn_fc2_w.astype(f32), mpnn_fc2_b.reshape(1, 256).astype(f32),
      fc_w.astype(f32), fc_b.reshape(1, hidden).astype(f32))

    return out.reshape(G, hidden)
